# baseline (speedup 1.0000x reference)
import functools

import jax
import jax.numpy as jnp
from jax import lax
from jax.experimental import pallas as pl
from jax.experimental.pallas import tpu as pltpu
from jax.experimental.pallas import tpu_sc as plsc

N_NODES = 10000
N_EDGES = 320000
NC, NS = 2, 16
NW = NC * NS
CHUNK = 128
N_CHUNKS = N_EDGES // CHUNK
CPW = 80
CPW_LAST = N_CHUNKS - (NW - 1) * CPW
GDEPTH = 8
PREF = 4
RPT = N_NODES // NS
ROW_BLK = 2000


@functools.lru_cache(maxsize=None)
def _make_segsum(d):
    mesh = plsc.VectorSubcoreMesh(core_axis_name="c", subcore_axis_name="s")

    @functools.partial(
        pl.kernel,
        out_type=jax.ShapeDtypeStruct((NC, N_NODES, d), jnp.float32),
        mesh=mesh,
        scratch_types=[
            pltpu.VMEM((CPW, CHUNK), jnp.int32),
            pltpu.VMEM((CPW, CHUNK), jnp.int32),
            pltpu.VMEM((GDEPTH, CHUNK, d), jnp.float32),
            pltpu.VMEM_SHARED((N_NODES, d), jnp.float32),
            pltpu.SemaphoreType.DMA((GDEPTH,)),
            pltpu.SemaphoreType.DMA((GDEPTH,)),
        ],
        compiler_params=pltpu.CompilerParams(use_tc_tiling_on_sc=False),
    )
    def segsum(table, srcc, dstc, zrows, out, src_v, dst_v, rows_v, acc,
               gsem, ssem):
        c = lax.axis_index("c")
        s = lax.axis_index("s")
        wid = c * NS + s
        last = wid == NW - 1
        nchunks = jnp.where(last, CPW_LAST, CPW)
        pltpu.sync_copy(zrows.at[pl.ds(s * RPT, RPT)],
                        acc.at[pl.ds(s * RPT, RPT)])

        @pl.when(jnp.logical_not(last))
        def _():
            pltpu.sync_copy(srcc.at[pl.ds(wid * CPW, CPW)], src_v)
            pltpu.sync_copy(dstc.at[pl.ds(wid * CPW, CPW)], dst_v)

        @pl.when(last)
        def _():
            pltpu.sync_copy(srcc.at[pl.ds(wid * CPW, CPW_LAST)],
                            src_v.at[pl.ds(0, CPW_LAST)])
            pltpu.sync_copy(dstc.at[pl.ds(wid * CPW, CPW_LAST)],
                            dst_v.at[pl.ds(0, CPW_LAST)])

        plsc.subcore_barrier()

        def gather(cc, b):
            pltpu.async_copy(table.at[src_v.at[cc]], rows_v.at[b], gsem.at[b])

        def gather_wait(cc, b):
            pltpu.make_async_copy(table.at[src_v.at[cc]], rows_v.at[b],
                                  gsem.at[b]).wait()

        def scatter(cc, b):
            pltpu.async_copy(rows_v.at[b], acc.at[dst_v.at[cc]], ssem.at[b],
                             add=True)

        def scatter_wait(cc, b):
            pltpu.make_async_copy(rows_v.at[b], acc.at[dst_v.at[cc]],
                                  ssem.at[b]).wait()

        for b in range(PREF):
            gather(b, b)

        def body(i, carry):
            j = i * GDEPTH
            for b in range(GDEPTH):
                cc = j + b
                bnxt = (b + PREF) % GDEPTH

                @pl.when(cc < nchunks)
                def _():
                    gather_wait(cc, b)
                    scatter(cc, b)

                prev = cc - PREF

                @pl.when(jnp.logical_and(prev >= 0, prev < nchunks))
                def _():
                    scatter_wait(prev, bnxt)

                nxt = cc + PREF

                @pl.when(nxt < nchunks)
                def _():
                    gather(nxt, bnxt)
            return carry

        nsteps = (nchunks + GDEPTH - 1) // GDEPTH + 1
        lax.fori_loop(0, nsteps, body, 0)
        plsc.subcore_barrier()
        pltpu.sync_copy(acc.at[pl.ds(s * RPT, RPT)],
                        out.at[c].at[pl.ds(s * RPT, RPT)])

    return segsum


def _segment_sum_sc(table, srcc, dstc, zrows):
    return _make_segsum(table.shape[1])(table, srcc, dstc, zrows)


def _proj_body(x_ref, w_ref, o_ref):
    o_ref[...] = jnp.dot(x_ref[...], w_ref[...],
                         preferred_element_type=jnp.float32)


def _proj(x, w):
    n, k = x.shape
    m = w.shape[1]
    return pl.pallas_call(
        _proj_body,
        grid=(n // ROW_BLK,),
        in_specs=[pl.BlockSpec((ROW_BLK, k), lambda i: (i, 0)),
                  pl.BlockSpec((k, m), lambda i: (0, 0))],
        out_specs=pl.BlockSpec((ROW_BLK, m), lambda i: (i, 0)),
        out_shape=jax.ShapeDtypeStruct((n, m), jnp.float32),
    )(x, w)


def _mid_body(agg_ref, x_ref, wroot_ref, b1_ref, wl1_ref, bl1_ref,
              w2rel_ref, w2root_ref, b2_ref, p2_ref, r2_ref):
    h = (agg_ref[0] + agg_ref[1] + b1_ref[...]
         + jnp.dot(x_ref[...], wroot_ref[...],
                   preferred_element_type=jnp.float32))
    t = jax.nn.relu(jnp.dot(h, wl1_ref[...],
                            preferred_element_type=jnp.float32) + bl1_ref[...])
    p2_ref[...] = jnp.dot(t, w2rel_ref[...],
                          preferred_element_type=jnp.float32)
    r2_ref[...] = (jnp.dot(t, w2root_ref[...],
                           preferred_element_type=jnp.float32) + b2_ref[...])


def _mid(agg1, x, w1_root, b1, wl1, bl1, w2_rel, w2_root, b2):
    n = x.shape[0]
    full = lambda shape: pl.BlockSpec(shape, lambda i: tuple(0 for _ in shape))
    row = lambda m: pl.BlockSpec((ROW_BLK, m), lambda i: (i, 0))
    return pl.pallas_call(
        _mid_body,
        grid=(n // ROW_BLK,),
        in_specs=[pl.BlockSpec((NC, ROW_BLK, 64), lambda i: (0, i, 0)),
                  row(128), full((128, 64)), full((1, 64)),
                  full((64, 32)), full((1, 32)), full((32, 16)),
                  full((32, 16)), full((1, 16))],
        out_specs=[row(16), row(16)],
        out_shape=[jax.ShapeDtypeStruct((n, 16), jnp.float32),
                   jax.ShapeDtypeStruct((n, 16), jnp.float32)],
    )(agg1, x, w1_root, b1.reshape(1, 64), wl1, bl1.reshape(1, 32),
      w2_rel, w2_root, b2.reshape(1, 16))


def _final_body(agg_ref, r2_ref, wl2_ref, bl2_ref, o_ref):
    h2 = agg_ref[0] + agg_ref[1] + r2_ref[...]
    logits = jnp.dot(h2, wl2_ref[...],
                     preferred_element_type=jnp.float32) + bl2_ref[...]
    m = jnp.max(logits, axis=1, keepdims=True)
    sh = logits - m
    lse = jnp.log(jnp.sum(jnp.exp(sh), axis=1, keepdims=True))
    o_ref[...] = sh - lse


def _final(agg2, r2, wl2, bl2):
    n = r2.shape[0]
    ncls = wl2.shape[1]
    full = lambda shape: pl.BlockSpec(shape, lambda i: tuple(0 for _ in shape))
    row = lambda m: pl.BlockSpec((ROW_BLK, m), lambda i: (i, 0))
    return pl.pallas_call(
        _final_body,
        grid=(n // ROW_BLK,),
        in_specs=[pl.BlockSpec((NC, ROW_BLK, 16), lambda i: (0, i, 0)),
                  row(16), full((16, ncls)), full((1, ncls))],
        out_specs=row(ncls),
        out_shape=jax.ShapeDtypeStruct((n, ncls), jnp.float32),
    )(agg2, r2, wl2, bl2.reshape(1, ncls))


def kernel(x, edge_index, W1_rel, b1, W1_root, Wl1, bl1, W2_rel, b2, W2_root,
           Wl2, bl2):
    srcc = edge_index[0].astype(jnp.int32).reshape(N_CHUNKS, CHUNK)
    dstc = edge_index[1].astype(jnp.int32).reshape(N_CHUNKS, CHUNK)
    z64 = jnp.zeros((N_NODES, 64), jnp.float32)
    z16 = jnp.zeros((N_NODES, 16), jnp.float32)

    p1 = _proj(x, W1_rel)
    agg1 = _segment_sum_sc(p1, srcc, dstc, z64)
    p2, r2 = _mid(agg1, x, W1_root, b1, Wl1, bl1,
                  W2_rel, W2_root, b2)
    agg2 = _segment_sum_sc(p2, srcc, dstc, z16)
    return _final(agg2, r2, Wl2, bl2)
